# baseline (speedup 1.0000x reference)
import functools

import jax
import jax.numpy as jnp
from jax import lax
from jax.experimental import pallas as pl
from jax.experimental.pallas import tpu as pltpu
from jax.experimental.pallas import tpu_sc as plsc

_N = 10000
_E = 320000
_D = 128
_F = 512
_NP = 10240
_NC = 2
_NS = 16
_NW = _NC * _NS
_CH = 128
_JC = 80
_EP = _NW * _JC * _CH
_NROW = _EP // _CH
_NBUF = 2
_HC = _JC // 2
_JCV = _EP // (_NS * _CH)
_RPT = _NP // _NS
_RB = 1024
_GR = _NP // _RB
_SB = 1000
_SG = _N // _SB


def _deg_body(dst1, dst2, ones_h, zeros_h, o00, o01, o10, o11,
              idx1_v, idx2_v, ones_v, vbuf, deg1_sh, deg2_sh, sem):
  cid = lax.axis_index("c")
  sid = lax.axis_index("s")
  wid = sid * _NC + cid
  pltpu.sync_copy(zeros_h, vbuf)
  pltpu.sync_copy(vbuf, deg1_sh.at[pl.ds(sid * _RPT, _RPT)])
  pltpu.sync_copy(vbuf, deg2_sh.at[pl.ds(sid * _RPT, _RPT)])
  pltpu.sync_copy(ones_h, ones_v)
  plsc.subcore_barrier()
  row0 = wid * _JC
  pltpu.sync_copy(dst1.at[pl.ds(row0, _JC)], idx1_v)
  pltpu.sync_copy(dst2.at[pl.ds(row0, _JC)], idx2_v)
  for acc, idx_v in ((deg1_sh, idx1_v), (deg2_sh, idx2_v)):

    @pl.loop(0, _JC)
    def _(j):
      pltpu.async_copy(ones_v, acc.at[idx_v.at[j]], sem, add=True)

  @pl.loop(0, 2 * _JC)
  def _(j):
    pltpu.make_async_copy(ones_h, ones_v, sem).wait()

  plsc.subcore_barrier()

  @pl.when(cid == 0)
  def _():
    pltpu.sync_copy(deg1_sh.at[pl.ds(sid * _RPT, _RPT)], vbuf)
    pltpu.sync_copy(vbuf, o00.at[pl.ds(sid * _RPT, _RPT)])
    pltpu.sync_copy(deg2_sh.at[pl.ds(sid * _RPT, _RPT)], vbuf)
    pltpu.sync_copy(vbuf, o10.at[pl.ds(sid * _RPT, _RPT)])

  @pl.when(cid == 1)
  def _():
    pltpu.sync_copy(deg1_sh.at[pl.ds(sid * _RPT, _RPT)], vbuf)
    pltpu.sync_copy(vbuf, o01.at[pl.ds(sid * _RPT, _RPT)])
    pltpu.sync_copy(deg2_sh.at[pl.ds(sid * _RPT, _RPT)], vbuf)
    pltpu.sync_copy(vbuf, o11.at[pl.ds(sid * _RPT, _RPT)])


_deg_call = functools.partial(
    pl.kernel,
    out_type=[jax.ShapeDtypeStruct((_NP,), jnp.float32)] * 4,
    mesh=plsc.VectorSubcoreMesh(core_axis_name="c", subcore_axis_name="s"),
    scratch_types=[
        pltpu.VMEM((_JC, _CH), jnp.int32),
        pltpu.VMEM((_JC, _CH), jnp.int32),
        pltpu.VMEM((_CH,), jnp.float32),
        pltpu.VMEM((_RPT,), jnp.float32),
        pltpu.VMEM_SHARED((_NP,), jnp.float32),
        pltpu.VMEM_SHARED((_NP,), jnp.float32),
        pltpu.SemaphoreType.DMA,
    ],
)(_deg_body)


def _agg_body(yh, sh, dh, zeros2d, out,
              src_v, dst_v, rb0, rb1, acc_sh, sem0, sem1):
  cid = lax.axis_index("c")
  sid = lax.axis_index("s")
  wid = sid * _NC + cid
  rbufs = (rb0, rb1)
  sems = (sem0, sem1)
  pltpu.sync_copy(zeros2d, rb0)
  for k in range(_RPT // _CH):
    pltpu.sync_copy(rb0, acc_sh.at[pl.ds(sid * _RPT + k * _CH, _CH)])
  plsc.subcore_barrier()
  for h in range(_JC // _HC):
    pltpu.sync_copy(sh.at[pl.ds(wid * _JC + h * _HC, _HC)], src_v)
    pltpu.sync_copy(dh.at[pl.ds(wid * _JC + h * _HC, _HC)], dst_v)
    for b in range(_NBUF):
      pltpu.async_copy(yh.at[src_v.at[b]], rbufs[b], sems[b])

    @pl.loop(0, _HC - _NBUF, step=_NBUF)
    def _(j):
      for b in range(_NBUF):
        pltpu.make_async_copy(yh.at[pl.ds(0, _CH)], rbufs[b],
                              sems[b]).wait()
        pltpu.sync_copy(rbufs[b], acc_sh.at[dst_v.at[j + b]], add=True)
        pltpu.async_copy(yh.at[src_v.at[j + b + _NBUF]], rbufs[b], sems[b])

    for b in range(_NBUF):
      pltpu.make_async_copy(yh.at[pl.ds(0, _CH)], rbufs[b], sems[b]).wait()
      pltpu.sync_copy(rbufs[b], acc_sh.at[dst_v.at[_HC - _NBUF + b]],
                      add=True)
  plsc.subcore_barrier()
  for k in range(_RPT // _CH):
    pltpu.sync_copy(acc_sh.at[pl.ds(sid * _RPT + k * _CH, _CH)], rb0)
    pltpu.sync_copy(rb0, out.at[cid, pl.ds(sid * _RPT + k * _CH, _CH)])


_agg_call = functools.partial(
    pl.kernel,
    out_type=jax.ShapeDtypeStruct((_NC, _NP, _D), jnp.float32),
    mesh=plsc.VectorSubcoreMesh(core_axis_name="c", subcore_axis_name="s"),
    scratch_types=[
        pltpu.VMEM((_HC, _CH), jnp.int32),
        pltpu.VMEM((_HC, _CH), jnp.int32),
        pltpu.VMEM((_CH, _D), jnp.float32),
        pltpu.VMEM((_CH, _D), jnp.float32),
        pltpu.VMEM_SHARED((_NP, _D), jnp.float32),
        pltpu.SemaphoreType.DMA,
        pltpu.SemaphoreType.DMA,
    ],
)(_agg_body)


def _prep_body(dp_ref, xs_ref, ys_ref):
  dp = dp_ref[0]
  deg = jnp.sum(dp, axis=1, keepdims=True) + 1.0
  dis = lax.rsqrt(deg)
  bc = jnp.broadcast_to(dis, (_RB, _D))
  ys_ref[0] = xs_ref[0] * bc


def _prep_call(dp_t, xs):
  return pl.pallas_call(
      _prep_body,
      grid=(2, _GR),
      in_specs=[
          pl.BlockSpec((1, _RB, 2), lambda v, r: (v, r, 0)),
          pl.BlockSpec((1, _RB, _D), lambda v, r: (v, r, 0)),
      ],
      out_specs=pl.BlockSpec((1, _RB, _D), lambda v, r: (v, r, 0)),
      out_shape=jax.ShapeDtypeStruct((2, _NP, _D), jnp.float32),
      compiler_params=pltpu.CompilerParams(
          dimension_semantics=("arbitrary", "arbitrary")),
  )(dp_t, xs)


def _stats_out_body(dp_ref, sp_ref, ys_ref, w_ref, z_ref,
                    agg_s, cs_s, gram_s, inv_s):
  r = pl.program_id(0)

  @pl.when(r < _SG)
  def _():
    dp = dp_ref[0]
    deg = jnp.sum(dp, axis=1, keepdims=True) + 1.0
    dis = lax.rsqrt(deg)
    bc = jnp.broadcast_to(dis, (_SB, _D))
    a = bc * (sp_ref[0] + sp_ref[1] + ys_ref[0])
    agg_s[pl.ds(r * _SB, _SB), :] = a

    @pl.when(r == 0)
    def _():
      cs_s[...] = jnp.zeros_like(cs_s)
      gram_s[...] = jnp.zeros_like(gram_s)

    cs_s[...] += jnp.sum(a, axis=0, keepdims=True)
    gram_s[...] += lax.dot_general(a, a, (((0,), (0,)), ((), ())),
                                   preferred_element_type=jnp.float32)

  @pl.when(r == _SG - 1)
  def _():
    mu = cs_s[...] * (1.0 / _N)
    outer = lax.dot_general(mu, mu, (((0,), (0,)), ((), ())),
                            precision=lax.Precision.HIGHEST)
    gc = gram_s[...] - _N * outer
    w = w_ref[...]
    gw = lax.dot_general(gc, w, (((1,), (0,)), ((), ())),
                         precision=lax.Precision.HIGHEST)
    var = jnp.sum(w * gw, axis=0, keepdims=True) * (1.0 / (_N - 1))
    inv_s[...] = lax.rsqrt(var)

  @pl.when(r >= _SG)
  def _():
    ro = r - _SG
    mu = cs_s[...] * (1.0 / _N)
    a = agg_s[pl.ds(ro * _SB, _SB), :] - mu
    z_ref[...] = lax.dot_general(
        a, w_ref[...], (((1,), (0,)), ((), ())),
        preferred_element_type=jnp.float32) * inv_s[...]


def _stats_out_call(dp_t, sp, ys, w, view):
  return pl.pallas_call(
      _stats_out_body,
      grid=(2 * _SG,),
      in_specs=[
          pl.BlockSpec((1, _SB, 2),
                       lambda r: (view, jnp.minimum(r, _SG - 1), 0)),
          pl.BlockSpec((_NC, _SB, _D),
                       lambda r: (0, jnp.minimum(r, _SG - 1), 0)),
          pl.BlockSpec((1, _SB, _D),
                       lambda r: (view, jnp.minimum(r, _SG - 1), 0)),
          pl.BlockSpec((_D, _F), lambda r: (0, 0)),
      ],
      out_specs=pl.BlockSpec(
          (_SB, _F), lambda r: (jnp.where(r < _SG, 0, r - _SG), 0)),
      out_shape=jax.ShapeDtypeStruct((_N, _F), jnp.float32),
      scratch_shapes=[
          pltpu.VMEM((_N, _D), jnp.float32),
          pltpu.VMEM((1, _D), jnp.float32),
          pltpu.VMEM((_D, _D), jnp.float32),
          pltpu.VMEM((1, _F), jnp.float32),
      ],
      compiler_params=pltpu.CompilerParams(
          dimension_semantics=("arbitrary",)),
  )(dp_t, sp, ys, w)


def _pad_edges(idx):
  fill = _N + (jnp.arange(_EP - _E, dtype=jnp.int32) % (_NP - _N))
  return jnp.concatenate([idx, fill]).reshape(_NROW, _CH)


def kernel(x_1, edge_index_1, x_2, edge_index_2, W, b):
  del b
  src1 = _pad_edges(edge_index_1[0])
  dst1 = _pad_edges(edge_index_1[1])
  src2 = _pad_edges(edge_index_2[0])
  dst2 = _pad_edges(edge_index_2[1])
  pad = ((0, _NP - _N), (0, 0))
  xs = jnp.stack([jnp.pad(x_1, pad), jnp.pad(x_2, pad)])

  ones_c = jnp.ones((_CH,), jnp.float32)
  zeros_r = jnp.zeros((_RPT,), jnp.float32)
  zeros2d = jnp.zeros((_CH, _D), jnp.float32)

  d00, d01, d10, d11 = _deg_call(dst1, dst2, ones_c, zeros_r)
  dp_t = jnp.stack([jnp.stack([d00, d01], axis=-1),
                    jnp.stack([d10, d11], axis=-1)])
  ys = _prep_call(dp_t, xs)
  s1 = _agg_call(ys[0], src1, dst1, zeros2d)
  s2 = _agg_call(ys[1], src2, dst2, zeros2d)
  z1 = _stats_out_call(dp_t, s1, ys, W, 0)
  z2 = _stats_out_call(dp_t, s2, ys, W, 1)
  return z1, z2


# speedup vs baseline: 1.0087x; 1.0087x over previous
import functools

import jax
import jax.numpy as jnp
from jax import lax
from jax.experimental import pallas as pl
from jax.experimental.pallas import tpu as pltpu
from jax.experimental.pallas import tpu_sc as plsc

_N = 10000
_E = 320000
_D = 128
_F = 512
_NP = 10240
_NC = 2
_NS = 16
_NW = _NC * _NS
_CH = 128
_JC = 80
_EP = _NW * _JC * _CH
_NROW = _EP // _CH
_NBUF = 2
_HC = _JC // 2
_JCV = _EP // (_NS * _CH)
_RPT = _NP // _NS
_RB = 1024
_GR = _NP // _RB
_SB = 1000
_SG = _N // _SB


def _deg_body(dst1, dst2, ones_h, zeros_h, o00, o01, o10, o11,
              idx1_v, idx2_v, ones_v, vbuf, deg1_sh, deg2_sh, sem):
  cid = lax.axis_index("c")
  sid = lax.axis_index("s")
  wid = sid * _NC + cid
  pltpu.sync_copy(zeros_h, vbuf)
  pltpu.sync_copy(vbuf, deg1_sh.at[pl.ds(sid * _RPT, _RPT)])
  pltpu.sync_copy(vbuf, deg2_sh.at[pl.ds(sid * _RPT, _RPT)])
  pltpu.sync_copy(ones_h, ones_v)
  plsc.subcore_barrier()
  row0 = wid * _JC
  pltpu.sync_copy(dst1.at[pl.ds(row0, _JC)], idx1_v)
  pltpu.sync_copy(dst2.at[pl.ds(row0, _JC)], idx2_v)
  for acc, idx_v in ((deg1_sh, idx1_v), (deg2_sh, idx2_v)):

    @pl.loop(0, _JC)
    def _(j):
      pltpu.async_copy(ones_v, acc.at[idx_v.at[j]], sem, add=True)

  @pl.loop(0, 2 * _JC)
  def _(j):
    pltpu.make_async_copy(ones_h, ones_v, sem).wait()

  plsc.subcore_barrier()

  @pl.when(cid == 0)
  def _():
    pltpu.sync_copy(deg1_sh.at[pl.ds(sid * _RPT, _RPT)], vbuf)
    pltpu.sync_copy(vbuf, o00.at[pl.ds(sid * _RPT, _RPT)])
    pltpu.sync_copy(deg2_sh.at[pl.ds(sid * _RPT, _RPT)], vbuf)
    pltpu.sync_copy(vbuf, o10.at[pl.ds(sid * _RPT, _RPT)])

  @pl.when(cid == 1)
  def _():
    pltpu.sync_copy(deg1_sh.at[pl.ds(sid * _RPT, _RPT)], vbuf)
    pltpu.sync_copy(vbuf, o01.at[pl.ds(sid * _RPT, _RPT)])
    pltpu.sync_copy(deg2_sh.at[pl.ds(sid * _RPT, _RPT)], vbuf)
    pltpu.sync_copy(vbuf, o11.at[pl.ds(sid * _RPT, _RPT)])


_deg_call = functools.partial(
    pl.kernel,
    out_type=[jax.ShapeDtypeStruct((_NP,), jnp.float32)] * 4,
    mesh=plsc.VectorSubcoreMesh(core_axis_name="c", subcore_axis_name="s"),
    scratch_types=[
        pltpu.VMEM((_JC, _CH), jnp.int32),
        pltpu.VMEM((_JC, _CH), jnp.int32),
        pltpu.VMEM((_CH,), jnp.float32),
        pltpu.VMEM((_RPT,), jnp.float32),
        pltpu.VMEM_SHARED((_NP,), jnp.float32),
        pltpu.VMEM_SHARED((_NP,), jnp.float32),
        pltpu.SemaphoreType.DMA,
    ],
)(_deg_body)


def _agg_body(y1, y2, s1h, d1h, s2h, d2h, zeros2d, out,
              src_v, dst_v, rb0, rb1, acc_sh, sem0, sem1):
  cid = lax.axis_index("c")
  sid = lax.axis_index("s")
  rbufs = (rb0, rb1)
  sems = (sem0, sem1)
  pltpu.sync_copy(zeros2d, rb0)
  for k in range(_RPT // _CH):
    pltpu.sync_copy(rb0, acc_sh.at[pl.ds(sid * _RPT + k * _CH, _CH)])
  plsc.subcore_barrier()

  def pump(yh, sh, dh):
    for h in range(_JCV // _HC):
      pltpu.sync_copy(sh.at[pl.ds(sid * _JCV + h * _HC, _HC)], src_v)
      pltpu.sync_copy(dh.at[pl.ds(sid * _JCV + h * _HC, _HC)], dst_v)
      for b in range(_NBUF):
        pltpu.async_copy(yh.at[src_v.at[b]], rbufs[b], sems[b])

      @pl.loop(0, _HC - _NBUF, step=_NBUF)
      def _(j):
        for b in range(_NBUF):
          pltpu.make_async_copy(yh.at[pl.ds(0, _CH)], rbufs[b],
                                sems[b]).wait()
          pltpu.sync_copy(rbufs[b], acc_sh.at[dst_v.at[j + b]], add=True)
          pltpu.async_copy(yh.at[src_v.at[j + b + _NBUF]], rbufs[b], sems[b])

      for b in range(_NBUF):
        pltpu.make_async_copy(yh.at[pl.ds(0, _CH)], rbufs[b], sems[b]).wait()
        pltpu.sync_copy(rbufs[b], acc_sh.at[dst_v.at[_HC - _NBUF + b]],
                        add=True)

  @pl.when(cid == 0)
  def _():
    pump(y1, s1h, d1h)

  @pl.when(cid == 1)
  def _():
    pump(y2, s2h, d2h)

  plsc.subcore_barrier()

  def dump(v):
    for k in range(_RPT // _CH):
      pltpu.sync_copy(acc_sh.at[pl.ds(sid * _RPT + k * _CH, _CH)], rb0)
      pltpu.sync_copy(rb0, out.at[v, pl.ds(sid * _RPT + k * _CH, _CH)])

  @pl.when(cid == 0)
  def _():
    dump(0)

  @pl.when(cid == 1)
  def _():
    dump(1)


_agg_call = functools.partial(
    pl.kernel,
    out_type=jax.ShapeDtypeStruct((2, _NP, _D), jnp.float32),
    mesh=plsc.VectorSubcoreMesh(core_axis_name="c", subcore_axis_name="s"),
    scratch_types=[
        pltpu.VMEM((_HC, _CH), jnp.int32),
        pltpu.VMEM((_HC, _CH), jnp.int32),
        pltpu.VMEM((_CH, _D), jnp.float32),
        pltpu.VMEM((_CH, _D), jnp.float32),
        pltpu.VMEM_SHARED((_NP, _D), jnp.float32),
        pltpu.SemaphoreType.DMA,
        pltpu.SemaphoreType.DMA,
    ],
)(_agg_body)


def _prep_body(dp_ref, xs_ref, ys_ref):
  dp = dp_ref[0]
  deg = jnp.sum(dp, axis=1, keepdims=True) + 1.0
  dis = lax.rsqrt(deg)
  bc = jnp.broadcast_to(dis, (_RB, _D))
  ys_ref[0] = xs_ref[0] * bc


def _prep_call(dp_t, xs):
  return pl.pallas_call(
      _prep_body,
      grid=(2, _GR),
      in_specs=[
          pl.BlockSpec((1, _RB, 2), lambda v, r: (v, r, 0)),
          pl.BlockSpec((1, _RB, _D), lambda v, r: (v, r, 0)),
      ],
      out_specs=pl.BlockSpec((1, _RB, _D), lambda v, r: (v, r, 0)),
      out_shape=jax.ShapeDtypeStruct((2, _NP, _D), jnp.float32),
      compiler_params=pltpu.CompilerParams(
          dimension_semantics=("arbitrary", "arbitrary")),
  )(dp_t, xs)


def _stats_out_body(dp_ref, sp_ref, ys_ref, w_ref, z1_ref, z2_ref,
                    agg_s, cs_s, gram_s, inv_s):
  v = pl.program_id(0)
  r = pl.program_id(1)

  @pl.when(r < _SG)
  def _():
    dp = dp_ref[0]
    deg = jnp.sum(dp, axis=1, keepdims=True) + 1.0
    dis = lax.rsqrt(deg)
    bc = jnp.broadcast_to(dis, (_SB, _D))
    a = bc * (sp_ref[0] + ys_ref[0])
    agg_s[pl.ds(r * _SB, _SB), :] = a

    @pl.when(r == 0)
    def _():
      cs_s[...] = jnp.zeros_like(cs_s)
      gram_s[...] = jnp.zeros_like(gram_s)

    cs_s[...] += jnp.sum(a, axis=0, keepdims=True)
    gram_s[...] += lax.dot_general(a, a, (((0,), (0,)), ((), ())),
                                   preferred_element_type=jnp.float32)

  @pl.when(r == _SG - 1)
  def _():
    mu = cs_s[...] * (1.0 / _N)
    outer = lax.dot_general(mu, mu, (((0,), (0,)), ((), ())),
                            precision=lax.Precision.HIGHEST)
    gc = gram_s[...] - _N * outer
    w = w_ref[...]
    gw = lax.dot_general(gc, w, (((1,), (0,)), ((), ())),
                         precision=lax.Precision.HIGHEST)
    var = jnp.sum(w * gw, axis=0, keepdims=True) * (1.0 / (_N - 1))
    inv_s[...] = lax.rsqrt(var)

  @pl.when(r >= _SG)
  def _():
    ro = r - _SG
    mu = cs_s[...] * (1.0 / _N)
    a = agg_s[pl.ds(ro * _SB, _SB), :] - mu
    z = lax.dot_general(a, w_ref[...], (((1,), (0,)), ((), ())),
                        preferred_element_type=jnp.float32) * inv_s[...]

    @pl.when(v == 0)
    def _():
      z1_ref[...] = z

    @pl.when(v == 1)
    def _():
      z2_ref[...] = z


def _stats_out_call(dp_t, s_parts, ys, w):
  return pl.pallas_call(
      _stats_out_body,
      grid=(2, 2 * _SG),
      in_specs=[
          pl.BlockSpec((1, _SB, 2),
                       lambda v, r: (v, jnp.minimum(r, _SG - 1), 0)),
          pl.BlockSpec((1, _SB, _D),
                       lambda v, r: (v, jnp.minimum(r, _SG - 1), 0)),
          pl.BlockSpec((1, _SB, _D),
                       lambda v, r: (v, jnp.minimum(r, _SG - 1), 0)),
          pl.BlockSpec((_D, _F), lambda v, r: (0, 0)),
      ],
      out_specs=[
          pl.BlockSpec(
              (_SB, _F),
              lambda v, r: (jnp.where(v == 0,
                                      jnp.where(r < _SG, 0, r - _SG),
                                      _SG - 1), 0)),
          pl.BlockSpec(
              (_SB, _F),
              lambda v, r: (jnp.where(v == 1,
                                      jnp.where(r < _SG, 0, r - _SG),
                                      0), 0)),
      ],
      out_shape=[
          jax.ShapeDtypeStruct((_N, _F), jnp.float32),
          jax.ShapeDtypeStruct((_N, _F), jnp.float32),
      ],
      scratch_shapes=[
          pltpu.VMEM((_N, _D), jnp.float32),
          pltpu.VMEM((1, _D), jnp.float32),
          pltpu.VMEM((_D, _D), jnp.float32),
          pltpu.VMEM((1, _F), jnp.float32),
      ],
      compiler_params=pltpu.CompilerParams(
          dimension_semantics=("arbitrary", "arbitrary")),
  )(dp_t, s_parts, ys, w)


def _pad_edges(idx):
  fill = _N + (jnp.arange(_EP - _E, dtype=jnp.int32) % (_NP - _N))
  return jnp.concatenate([idx, fill]).reshape(_NROW, _CH)


def kernel(x_1, edge_index_1, x_2, edge_index_2, W, b):
  del b
  src1 = _pad_edges(edge_index_1[0])
  dst1 = _pad_edges(edge_index_1[1])
  src2 = _pad_edges(edge_index_2[0])
  dst2 = _pad_edges(edge_index_2[1])
  pad = ((0, _NP - _N), (0, 0))
  xs = jnp.stack([jnp.pad(x_1, pad), jnp.pad(x_2, pad)])

  ones_c = jnp.ones((_CH,), jnp.float32)
  zeros_r = jnp.zeros((_RPT,), jnp.float32)
  zeros2d = jnp.zeros((_CH, _D), jnp.float32)

  d00, d01, d10, d11 = _deg_call(dst1, dst2, ones_c, zeros_r)
  dp_t = jnp.stack([jnp.stack([d00, d01], axis=-1),
                    jnp.stack([d10, d11], axis=-1)])
  ys = _prep_call(dp_t, xs)
  s_parts = _agg_call(ys[0], ys[1], src1, dst1, src2, dst2, zeros2d)
  z1, z2 = _stats_out_call(dp_t, s_parts, ys, W)
  return z1, z2
